# baseline (speedup 1.0000x reference)
import functools

import jax
import jax.numpy as jnp
from jax import lax
from jax.experimental import pallas as pl
from jax.experimental.pallas import tpu as pltpu
from jax.experimental.pallas import tpu_sc as plsc

NC, NS = 2, 16
NW = NC * NS
BATCH = 16384
HIST = 50
D = 32
PB = BATCH // NW
NBT = PB // 128
FT = D // 8
TT = HIST // 2

_mesh = plsc.VectorSubcoreMesh(core_axis_name="c", subcore_axis_name="s")


@functools.partial(
    pl.kernel,
    out_type=jax.ShapeDtypeStruct((HIST, FT, BATCH // 128, 8, 128), jnp.float32),
    mesh=_mesh,
    scratch_types=[
        pltpu.VMEM((HIST, PB), jnp.int32),
        [pltpu.VMEM((NBT, 128, D), jnp.float32) for _ in range(2)],
        [pltpu.VMEM((FT, NBT, 8, 128), jnp.float32) for _ in range(2)],
        [pltpu.SemaphoreType.DMA for _ in range(2)],
        [pltpu.SemaphoreType.DMA for _ in range(2)],
    ],
    compiler_params=pltpu.CompilerParams(
        use_tc_tiling_on_sc=False, needs_layout_passes=False
    ),
)
def _lookup(idx_hbm, table_hbm, out_hbm, idx_v, gbufs, wbufs, gsems, wsems):
    wid = lax.axis_index("s") * NC + lax.axis_index("c")
    b0 = wid * PB
    bt0 = wid * NBT
    pltpu.sync_copy(idx_hbm.at[:, pl.ds(b0, PB)], idx_v)

    def gather_cps(h, slot):
        return [
            pltpu.make_async_copy(
                table_hbm.at[idx_v.at[h, pl.ds(bt * 128, 128)]],
                gbufs[slot].at[bt],
                gsems[slot],
            )
            for bt in range(NBT)
        ]

    def write_cps(h, slot):
        return [
            pltpu.make_async_copy(
                wbufs[slot].at[ft],
                out_hbm.at[h, ft, pl.ds(bt0, NBT)],
                wsems[slot],
            )
            for ft in range(FT)
        ]

    def start_all(cps):
        for cp in cps:
            cp.start()

    def wait_all(cps):
        for cp in cps:
            cp.wait()

    lane = lax.iota(jnp.int32, 16)
    bt_ids = [jnp.full((16,), bt, jnp.int32) for bt in range(NBT)]
    f_ids = [jnp.full((16,), f, jnp.int32) for f in range(D)]

    def transpose(slot):
        def c_body(c, carry):
            rows = lane + c * 16
            for bt in range(NBT):
                for ft in range(FT):
                    for f8 in range(8):
                        vals = plsc.load_gather(
                            gbufs[slot], [bt_ids[bt], rows, f_ids[ft * 8 + f8]]
                        )
                        wbufs[slot][ft, bt, f8, pl.ds(c * 16, 16)] = vals
            return carry
        lax.fori_loop(0, 8, c_body, 0)

    start_all(gather_cps(0, 0))

    def t_body(t, carry):
        for p in range(2):
            h = 2 * t + p
            slot, other = p, 1 - p

            if p == 0:
                pl.when(t > 0)(lambda: wait_all(write_cps(h - 1, other)))
                start_all(gather_cps(h + 1, other))
            else:
                def refill():
                    wait_all(write_cps(h - 1, other))
                    start_all(gather_cps(h + 1, other))
                pl.when(t < TT - 1)(refill)

            wait_all(gather_cps(h, slot))
            transpose(slot)
            start_all(write_cps(h, slot))
        return carry

    lax.fori_loop(0, TT, t_body, 0)

    wait_all(write_cps(HIST - 2, 0))
    wait_all(write_cps(HIST - 1, 1))


def kernel(indices, table):
    idx_t = indices.astype(jnp.int32).T
    out5 = _lookup(idx_t, table)
    out = out5.transpose(2, 4, 0, 1, 3)
    return out.reshape(BATCH, HIST, D)


# speedup vs baseline: 1.2841x; 1.2841x over previous
import functools

import jax
import jax.numpy as jnp
from jax import lax
from jax.experimental import pallas as pl
from jax.experimental.pallas import tpu as pltpu
from jax.experimental.pallas import tpu_sc as plsc

NC, NS = 2, 16
NW = NC * NS
BATCH = 16384
HIST = 50
D = 32
PB = BATCH // NW
NBT = PB // 128
FT = D // 8
TT = HIST // 2

_mesh = plsc.VectorSubcoreMesh(core_axis_name="c", subcore_axis_name="s")


@functools.partial(
    pl.kernel,
    out_type=jax.ShapeDtypeStruct((HIST, FT, BATCH // 128, 8, 128), jnp.float32),
    mesh=_mesh,
    scratch_types=[
        pltpu.VMEM((HIST, PB), jnp.int32),
        [pltpu.VMEM((NBT, 128, D), jnp.float32) for _ in range(2)],
        [pltpu.VMEM((FT, NBT, 8, 128), jnp.float32) for _ in range(2)],
        [pltpu.SemaphoreType.DMA for _ in range(2)],
        [pltpu.SemaphoreType.DMA for _ in range(2)],
    ],
    compiler_params=pltpu.CompilerParams(
        use_tc_tiling_on_sc=False, needs_layout_passes=False
    ),
)
def _lookup(idx_hbm, table_hbm, out_hbm, idx_v, gbufs, wbufs, gsems, wsems):
    wid = lax.axis_index("s") * NC + lax.axis_index("c")
    b0 = wid * PB
    bt0 = wid * NBT
    pltpu.sync_copy(idx_hbm.at[:, pl.ds(b0, PB)], idx_v)

    def gather_cps(h, slot):
        return [
            pltpu.make_async_copy(
                table_hbm.at[idx_v.at[h, pl.ds(bt * 128, 128)]],
                gbufs[slot].at[bt],
                gsems[slot],
            )
            for bt in range(NBT)
        ]

    def write_cps(h, slot):
        return [
            pltpu.make_async_copy(
                wbufs[slot].at[ft],
                out_hbm.at[h, ft, pl.ds(bt0, NBT)],
                wsems[slot],
            )
            for ft in range(FT)
        ]

    def start_all(cps):
        for cp in cps:
            cp.start()

    def wait_all(cps):
        for cp in cps:
            cp.wait()

    lane = lax.iota(jnp.int32, 16)
    bt_ids = [jnp.full((16,), bt, jnp.int32) for bt in range(NBT)]
    f_ids = [jnp.full((16,), f, jnp.int32) for f in range(D)]

    def transpose(slot):
        def c_body(c, carry):
            rows = lane + c * 16
            for bt in range(NBT):
                for ft in range(FT):
                    vals = [
                        plsc.load_gather(
                            gbufs[slot], [bt_ids[bt], rows, f_ids[ft * 8 + f8]]
                        )
                        for f8 in range(8)
                    ]
                    for f8 in range(8):
                        wbufs[slot][ft, bt, f8, pl.ds(c * 16, 16)] = vals[f8]
            return carry
        lax.fori_loop(0, 8, c_body, 0)

    start_all(gather_cps(0, 0))

    def t_body(t, carry):
        for p in range(2):
            h = 2 * t + p
            slot, other = p, 1 - p

            if p == 0:
                pl.when(t > 0)(lambda: wait_all(write_cps(h - 1, other)))
                start_all(gather_cps(h + 1, other))
            else:
                def refill():
                    wait_all(write_cps(h - 1, other))
                    start_all(gather_cps(h + 1, other))
                pl.when(t < TT - 1)(refill)

            wait_all(gather_cps(h, slot))
            transpose(slot)
            start_all(write_cps(h, slot))
        return carry

    lax.fori_loop(0, TT, t_body, 0)

    wait_all(write_cps(HIST - 2, 0))
    wait_all(write_cps(HIST - 1, 1))


def kernel(indices, table):
    idx_t = indices.astype(jnp.int32).T
    out5 = _lookup(idx_t, table)
    out = out5.transpose(2, 4, 0, 1, 3)
    return out.reshape(BATCH, HIST, D)


# speedup vs baseline: 1.7188x; 1.3385x over previous
import functools

import jax
import jax.numpy as jnp
from jax import lax
from jax.experimental import pallas as pl
from jax.experimental.pallas import tpu as pltpu
from jax.experimental.pallas import tpu_sc as plsc

NC, NS = 2, 16
NW = NC * NS
BATCH = 16384
HIST = 50
D = 32
PB = BATCH // NW
NBT = PB // 128
FT = D // 8
TT = HIST // 2

_mesh = plsc.VectorSubcoreMesh(core_axis_name="c", subcore_axis_name="s")


@functools.partial(
    pl.kernel,
    out_type=jax.ShapeDtypeStruct((HIST, FT, BATCH // 128, 8, 128), jnp.float32),
    mesh=_mesh,
    scratch_types=[
        pltpu.VMEM((HIST, PB), jnp.int32),
        [pltpu.VMEM((NBT, 128, D), jnp.float32) for _ in range(2)],
        [pltpu.VMEM((FT, NBT, 8, 129), jnp.float32) for _ in range(2)],
        [pltpu.SemaphoreType.DMA for _ in range(2)],
        [pltpu.SemaphoreType.DMA for _ in range(2)],
    ],
    compiler_params=pltpu.CompilerParams(
        use_tc_tiling_on_sc=False, needs_layout_passes=False
    ),
)
def _lookup(idx_hbm, table_hbm, out_hbm, idx_v, gbufs, wbufs, gsems, wsems):
    wid = lax.axis_index("s") * NC + lax.axis_index("c")
    b0 = wid * PB
    bt0 = wid * NBT
    pltpu.sync_copy(idx_hbm.at[:, pl.ds(b0, PB)], idx_v)

    def gather_cps(h, slot):
        return [
            pltpu.make_async_copy(
                table_hbm.at[idx_v.at[h, pl.ds(bt * 128, 128)]],
                gbufs[slot].at[bt],
                gsems[slot],
            )
            for bt in range(NBT)
        ]

    def write_cps(h, slot):
        return [
            pltpu.make_async_copy(
                wbufs[slot].at[ft, :, :, pl.ds(0, 128)],
                out_hbm.at[h, ft, pl.ds(bt0, NBT)],
                wsems[slot],
            )
            for ft in range(FT)
        ]

    def start_all(cps):
        for cp in cps:
            cp.start()

    def wait_all(cps):
        for cp in cps:
            cp.wait()

    lane = lax.iota(jnp.int32, 16)
    ft_lo = lax.shift_right_logical(lane, 3)
    f8_id = lax.bitwise_and(lane, 7)
    bt_ids = [jnp.full((16,), bt, jnp.int32) for bt in range(NBT)]
    ft_his = ft_lo + 2
    zeros = jnp.zeros((16,), jnp.int32)

    def transpose(slot):
        def b_body(b, carry):
            bv = zeros + b
            for bt in range(NBT):
                lo = gbufs[slot][bt, b, pl.ds(0, 16)]
                hi = gbufs[slot][bt, b, pl.ds(16, 16)]
                plsc.store_scatter(
                    wbufs[slot], [ft_lo, bt_ids[bt], f8_id, bv], lo
                )
                plsc.store_scatter(
                    wbufs[slot], [ft_his, bt_ids[bt], f8_id, bv], hi
                )
            return carry
        lax.fori_loop(0, 128, b_body, 0)

    start_all(gather_cps(0, 0))

    def t_body(t, carry):
        for p in range(2):
            h = 2 * t + p
            slot, other = p, 1 - p

            if p == 0:
                pl.when(t > 0)(lambda: wait_all(write_cps(h - 1, other)))
                start_all(gather_cps(h + 1, other))
            else:
                def refill():
                    wait_all(write_cps(h - 1, other))
                    start_all(gather_cps(h + 1, other))
                pl.when(t < TT - 1)(refill)

            wait_all(gather_cps(h, slot))
            transpose(slot)
            start_all(write_cps(h, slot))
        return carry

    lax.fori_loop(0, TT, t_body, 0)

    wait_all(write_cps(HIST - 2, 0))
    wait_all(write_cps(HIST - 1, 1))


def kernel(indices, table):
    idx_t = indices.astype(jnp.int32).T
    out5 = _lookup(idx_t, table)
    out = out5.transpose(2, 4, 0, 1, 3)
    return out.reshape(BATCH, HIST, D)


# speedup vs baseline: 1.7321x; 1.0077x over previous
import functools

import jax
import jax.numpy as jnp
from jax import lax
from jax.experimental import pallas as pl
from jax.experimental.pallas import tpu as pltpu
from jax.experimental.pallas import tpu_sc as plsc

NC, NS = 2, 16
NW = NC * NS
BATCH = 16384
HIST = 50
D = 32
PB = BATCH // NW
NBT = PB // 128
FT = D // 8
TT = HIST // 2

_mesh = plsc.VectorSubcoreMesh(core_axis_name="c", subcore_axis_name="s")


@functools.partial(
    pl.kernel,
    out_type=jax.ShapeDtypeStruct((HIST, FT, BATCH // 128, 8, 128), jnp.float32),
    mesh=_mesh,
    scratch_types=[
        pltpu.VMEM((HIST, PB), jnp.int32),
        [pltpu.VMEM((NBT, 128, D), jnp.float32) for _ in range(2)],
        [pltpu.VMEM((FT, NBT, 8, 129), jnp.float32) for _ in range(2)],
        [pltpu.SemaphoreType.DMA for _ in range(2)],
        [pltpu.SemaphoreType.DMA for _ in range(2)],
    ],
    compiler_params=pltpu.CompilerParams(
        use_tc_tiling_on_sc=False, needs_layout_passes=False
    ),
)
def _lookup(idx_hbm, table_hbm, out_hbm, idx_v, gbufs, wbufs, gsems, wsems):
    wid = lax.axis_index("s") * NC + lax.axis_index("c")
    b0 = wid * PB
    bt0 = wid * NBT
    pltpu.sync_copy(idx_hbm.at[:, pl.ds(b0, PB)], idx_v)

    def gather_cps(h, slot):
        return [
            pltpu.make_async_copy(
                table_hbm.at[idx_v.at[h, pl.ds(bt * 128, 128)]],
                gbufs[slot].at[bt],
                gsems[slot],
            )
            for bt in range(NBT)
        ]

    def write_cps(h, slot):
        return [
            pltpu.make_async_copy(
                wbufs[slot].at[ft, :, :, pl.ds(0, 128)],
                out_hbm.at[h, ft, pl.ds(bt0, NBT)],
                wsems[slot],
            )
            for ft in range(FT)
        ]

    def start_all(cps):
        for cp in cps:
            cp.start()

    def wait_all(cps):
        for cp in cps:
            cp.wait()

    lane = lax.iota(jnp.int32, 16)
    ft_lo = lax.shift_right_logical(lane, 3)
    f8_id = lax.bitwise_and(lane, 7)
    bt_ids = [jnp.full((16,), bt, jnp.int32) for bt in range(NBT)]
    ft_his = ft_lo + 2
    zeros = jnp.zeros((16,), jnp.int32)

    def transpose(slot):
        def b_body(bi, carry):
            for b4 in range(4):
                b = bi * 4 + b4
                bv = zeros + b
                for bt in range(NBT):
                    lo = gbufs[slot][bt, b, pl.ds(0, 16)]
                    hi = gbufs[slot][bt, b, pl.ds(16, 16)]
                    plsc.store_scatter(
                        wbufs[slot], [ft_lo, bt_ids[bt], f8_id, bv], lo
                    )
                    plsc.store_scatter(
                        wbufs[slot], [ft_his, bt_ids[bt], f8_id, bv], hi
                    )
            return carry
        lax.fori_loop(0, 32, b_body, 0)

    start_all(gather_cps(0, 0))

    def t_body(t, carry):
        for p in range(2):
            h = 2 * t + p
            slot, other = p, 1 - p

            if p == 0:
                start_all(gather_cps(h + 1, other))
            else:
                pl.when(t < TT - 1)(
                    lambda: start_all(gather_cps(h + 1, other))
                )
            wait_all(gather_cps(h, slot))
            pl.when(t > 0)(lambda: wait_all(write_cps(h - 2, slot)))
            transpose(slot)
            start_all(write_cps(h, slot))
        return carry

    lax.fori_loop(0, TT, t_body, 0)

    wait_all(write_cps(HIST - 2, 0))
    wait_all(write_cps(HIST - 1, 1))


def kernel(indices, table):
    idx_t = indices.astype(jnp.int32).T
    out5 = _lookup(idx_t, table)
    out = out5.transpose(2, 4, 0, 1, 3)
    return out.reshape(BATCH, HIST, D)
